# baseline (speedup 1.0000x reference)
import jax
import jax.numpy as jnp
from jax.experimental import pallas as pl

N = 4096
NFEAT = 512
NHID = 128
NCLASS = 64
HEADS = 4
MASK_DT = jnp.float8_e4m3fn
LOG2E = 1.4426950408889634


def _castproj_body(x_ref, w_ref, adj_ref, m1_ref, o3, o0, o1, o2, b0, b1, b2):
    h = jnp.dot(x_ref[...], w_ref[...], preferred_element_type=jnp.float32)
    o3[...] = h[:, 0 * NHID:1 * NHID]
    hs = [h[:, 1 * NHID:2 * NHID], h[:, 2 * NHID:3 * NHID],
          h[:, 3 * NHID:4 * NHID]]
    for dst, src in zip([o0, o1, o2], hs):
        dst[...] = src
    for dst, src in zip([b0, b1, b2], hs):
        dst[...] = src.astype(jnp.bfloat16)
    m1_ref[...] = (adj_ref[...] > 0).astype(MASK_DT)


def _castproj(x, wcat, adj):
    BM = 512
    f32 = jax.ShapeDtypeStruct((N, NHID), jnp.float32)
    b16 = jax.ShapeDtypeStruct((N, NHID), jnp.bfloat16)
    blk = lambda i: (i, 0)
    return pl.pallas_call(
        _castproj_body,
        grid=(N // BM,),
        in_specs=[
            pl.BlockSpec((BM, NFEAT), blk),
            pl.BlockSpec((NFEAT, HEADS * NHID), lambda i: (0, 0)),
            pl.BlockSpec((BM, N), blk),
        ],
        out_specs=[pl.BlockSpec((BM, N), blk)] +
                  [pl.BlockSpec((BM, NHID), blk)] * 7,
        out_shape=[jax.ShapeDtypeStruct((N, N), MASK_DT)] +
                  [f32] * 4 + [b16] * 3,
    )(x, wcat, adj)


def _maskmm_body(a_ref, b_ref, o_ref):
    acc = jnp.dot(a_ref[...], b_ref[...], preferred_element_type=jnp.float32)
    o_ref[...] = (acc > 0).astype(MASK_DT)


def _maskmm(a, b):
    BM = 512
    return pl.pallas_call(
        _maskmm_body,
        grid=(N // BM,),
        in_specs=[
            pl.BlockSpec((BM, N), lambda i: (i, 0)),
            pl.BlockSpec((N, N), lambda i: (0, 0)),
        ],
        out_specs=pl.BlockSpec((BM, N), lambda i: (i, 0)),
        out_shape=jax.ShapeDtypeStruct((N, N), MASK_DT),
    )(a, b)


def _gatfinal_body(h3_ref, hf0, hf1, hf2, hb0, hb1, hb2, a_ref,
                   m0_ref, m1_ref, m2_ref, w_ref, b_ref, o_ref):
    i = pl.program_id(0)
    BM = h3_ref.shape[0]
    w = w_ref[...]
    acc = jnp.dot(jnp.maximum(h3_ref[...], 0.0), w[0:NHID, :],
                  preferred_element_type=jnp.float32)
    for k, (hf, hb, m_ref) in enumerate(
            [(hf0, hb0, m0_ref), (hf1, hb1, m1_ref), (hf2, hb2, m2_ref)]):
        hfull = hf[...]
        hblk = hf[pl.ds(i * BM, BM), :]
        a1 = a_ref[2 * k:2 * k + 1, :]
        a2 = a_ref[2 * k + 1:2 * k + 2, :]
        f1 = jnp.sum(hblk * a1, axis=1, keepdims=True)
        f2 = jnp.sum(hfull * a2, axis=1, keepdims=True)
        s = f1 + f2.T
        t = jnp.minimum(jnp.maximum(s, 0.2 * s), 80.0)
        p = jnp.exp2(t) * m_ref[...].astype(jnp.float32)
        denom = jnp.sum(p, axis=1, keepdims=True)
        g = jnp.dot(p.astype(jnp.bfloat16), hb[...],
                    preferred_element_type=jnp.float32) / denom
        acc += jnp.dot(jnp.maximum(g, 0.0),
                       w[NHID * (k + 1):NHID * (k + 2), :],
                       preferred_element_type=jnp.float32)
    logits = acc + b_ref[...]
    mx = jnp.max(logits, axis=1, keepdims=True)
    l = logits - mx
    lse = jnp.log(jnp.sum(jnp.exp(l), axis=1, keepdims=True))
    o_ref[...] = l - lse


def _gatfinal(h3, hs, hbs, a6, masks, fc_wt, fc_b2d):
    BM = 256
    full = lambda i: (0, 0)
    blk = lambda i: (i, 0)
    return pl.pallas_call(
        _gatfinal_body,
        grid=(N // BM,),
        in_specs=[pl.BlockSpec((BM, NHID), blk)] +
                 [pl.BlockSpec((N, NHID), full)] * 3 +
                 [pl.BlockSpec((N, NHID), full)] * 3 +
                 [pl.BlockSpec((2 * (HEADS - 1), NHID), full)] +
                 [pl.BlockSpec((BM, N), blk)] * 3 + [
            pl.BlockSpec((HEADS * NHID, NCLASS), full),
            pl.BlockSpec((1, NCLASS), full),
        ],
        out_specs=pl.BlockSpec((BM, NCLASS), blk),
        out_shape=jax.ShapeDtypeStruct((N, NCLASS), jnp.float32),
    )(h3, *hs, *hbs, a6, *masks, fc_wt, fc_b2d)


def kernel(x, adj, W, a, fc_w, fc_b):
    wcat = jnp.concatenate([W[HEADS - 1], W[0], W[1], W[2]], axis=1)
    m1, h3, h0, h1, h2, hb0, hb1, hb2 = _castproj(x, wcat, adj)
    m2 = _maskmm(m1, m1)
    m3 = _maskmm(m2, m1)
    a6 = (a.reshape(HEADS - 1, 2, NHID) * LOG2E).reshape(2 * (HEADS - 1), NHID)
    return _gatfinal(h3, [h0, h1, h2], [hb0, hb1, hb2], a6,
                     [m1, m2, m3], fc_w.T, fc_b.reshape(1, NCLASS))
